# baseline (speedup 1.0000x reference)
import functools

import jax
import jax.numpy as jnp
from jax import lax
from jax.experimental import pallas as pl
from jax.experimental.pallas import tpu as pltpu
from jax.experimental.pallas import tpu_sc as plsc

_NW = 32
_L = 16
_CHUNK = 2048
_FINE = 16
_COARSE_LVLS = 13
_FINE_LVLS = 4


def _make_resample(t_steps: int, n: int, d: int):
    n_coarse = n // _FINE
    n_chunks = n // _CHUNK
    vregs = _CHUNK // _L
    mesh = plsc.VectorSubcoreMesh(core_axis_name="c", subcore_axis_name="s")

    @functools.partial(
        pl.kernel,
        mesh=mesh,
        out_type=jax.ShapeDtypeStruct((t_steps, n, d), jnp.float32),
        scratch_types=[
            pltpu.VMEM((n_coarse,), jnp.float32),
            pltpu.VMEM((_CHUNK,), jnp.float32),
            pltpu.VMEM((_CHUNK,), jnp.int32),
            pltpu.VMEM((_CHUNK, _FINE), jnp.float32),
            pltpu.VMEM((_CHUNK,), jnp.int32),
            pltpu.VMEM((_CHUNK, d), jnp.float32),
            pltpu.SemaphoreType.DMA,
        ],
        compiler_params=pltpu.CompilerParams(
            use_tc_tiling_on_sc=False, needs_layout_passes=False),
    )
    def resample_k(coarse_hbm, q_hbm, cumrows_hbm, parts_hbm, out_hbm,
                   coarse_v, q_v, rid_v, rows_v, pid_v, prow_v, sem):
        wid = lax.axis_index("s") * 2 + lax.axis_index("c")
        iota = lax.broadcasted_iota(jnp.int32, (_L,), 0)

        pltpu.sync_copy(coarse_hbm.at[wid], coarse_v)

        def chunk_body(ci, carry):
            off = ci * _CHUNK
            pltpu.sync_copy(q_hbm.at[wid].at[pl.ds(off, _CHUNK)], q_v)

            def coarse_body(j, c):
                q = q_v[pl.ds(j * _L, _L)]
                cnt = jnp.zeros((_L,), jnp.int32)
                for lvl in range(_COARSE_LVLS):
                    step = 1 << (_COARSE_LVLS - 1 - lvl)
                    vals = plsc.load_gather(coarse_v, [cnt + (step - 1)])
                    cnt = cnt + jnp.where(vals < q, step, 0)
                rid_v[pl.ds(j * _L, _L)] = cnt
                return c

            lax.fori_loop(0, vregs, coarse_body, 0)

            pltpu.async_copy(cumrows_hbm.at[wid].at[rid_v], rows_v, sem).wait()

            def fine_body(j, c):
                q = q_v[pl.ds(j * _L, _L)]
                rid = rid_v[pl.ds(j * _L, _L)]
                row = j * _L + iota
                cnt = jnp.zeros((_L,), jnp.int32)
                for lvl in range(_FINE_LVLS):
                    step = 1 << (_FINE_LVLS - 1 - lvl)
                    vals = plsc.load_gather(rows_v, [row, cnt + (step - 1)])
                    cnt = cnt + jnp.where(vals < q, step, 0)
                pid_v[pl.ds(j * _L, _L)] = rid * _FINE + cnt
                return c

            lax.fori_loop(0, vregs, fine_body, 0)

            pltpu.async_copy(parts_hbm.at[wid].at[pid_v], prow_v, sem).wait()
            pltpu.sync_copy(prow_v, out_hbm.at[wid].at[pl.ds(off, _CHUNK)])
            return carry

        lax.fori_loop(0, n_chunks, chunk_body, 0)

    return resample_k


def kernel(particles, log_weights):
    t, n, d = particles.shape
    key = jax.random.key(42)
    keys = jax.random.split(key, t)

    def prep(lw, k):
        w = jnp.exp(lw)
        _, subkey = jax.random.split(k)
        p_cuml = jnp.cumsum(w)
        r = p_cuml[-1] * (1 - jax.random.uniform(subkey, (n,), dtype=p_cuml.dtype))
        return p_cuml, r

    p_cuml, r = jax.vmap(prep)(log_weights, keys)
    coarse = p_cuml[:, _FINE - 1::_FINE]
    cumrows = p_cuml.reshape(t, n // _FINE, _FINE)
    return _make_resample(t, n, d)(coarse, r, cumrows, particles)


# speedup vs baseline: 1.2716x; 1.2716x over previous
import functools

import jax
import jax.numpy as jnp
from jax import lax
from jax.experimental import pallas as pl
from jax.experimental.pallas import tpu as pltpu
from jax.experimental.pallas import tpu_sc as plsc

_L = 16
_CHUNK = 2048
_FINE = 16
_COARSE_LVLS = 13
_FINE_LVLS = 4
_BLK = 128


def _make_resample(t_steps: int, n: int, d: int):
    n_coarse = n // _FINE
    n_chunks = n // _CHUNK
    vregs = _CHUNK // _L
    nblk = n // _BLK
    blk_chunk = _CHUNK // _BLK
    mesh = plsc.VectorSubcoreMesh(core_axis_name="c", subcore_axis_name="s")

    @functools.partial(
        pl.kernel,
        mesh=mesh,
        out_type=jax.ShapeDtypeStruct((t_steps, nblk, d, _BLK), jnp.float32),
        scratch_types=[
            pltpu.VMEM((n_coarse,), jnp.float32),
            pltpu.VMEM((_CHUNK // _L, _L), jnp.float32),
            pltpu.VMEM((_CHUNK,), jnp.int32),
            pltpu.VMEM((_CHUNK, _FINE), jnp.float32),
            pltpu.VMEM((d, _CHUNK), jnp.int32),
            pltpu.VMEM((d, _CHUNK), jnp.float32),
            pltpu.VMEM((blk_chunk, d, _BLK), jnp.float32),
            pltpu.SemaphoreType.DMA,
        ],
        compiler_params=pltpu.CompilerParams(
            use_tc_tiling_on_sc=False, needs_layout_passes=False),
    )
    def resample_k(coarse_hbm, q_hbm, cumrows_hbm, parts_hbm, out_hbm,
                   coarse_v, q_v, rid_v, rows_v, idx_v, g_v, out_v, sem):
        wid = lax.axis_index("s") * 2 + lax.axis_index("c")
        iota = lax.broadcasted_iota(jnp.int32, (_L,), 0)

        pltpu.sync_copy(coarse_hbm.at[wid], coarse_v)

        def chunk_body(ci, carry):
            pltpu.sync_copy(
                q_hbm.at[wid].at[pl.ds(ci * (_CHUNK // _L), _CHUNK // _L)],
                q_v)

            def coarse_body(j, c):
                q = q_v[j]
                cnt = jnp.zeros((_L,), jnp.int32)
                for lvl in range(_COARSE_LVLS):
                    step = 1 << (_COARSE_LVLS - 1 - lvl)
                    vals = plsc.load_gather(coarse_v, [cnt + (step - 1)])
                    cnt = cnt + jnp.where(vals < q, step, 0)
                rid_v[pl.ds(j * _L, _L)] = cnt
                return c

            lax.fori_loop(0, vregs, coarse_body, 0)

            pltpu.async_copy(cumrows_hbm.at[wid].at[rid_v], rows_v, sem).wait()

            def fine_body(j, c):
                q = q_v[j]
                rid = rid_v[pl.ds(j * _L, _L)]
                row = j * _L + iota
                cnt = jnp.zeros((_L,), jnp.int32)
                for lvl in range(_FINE_LVLS):
                    step = 1 << (_FINE_LVLS - 1 - lvl)
                    vals = plsc.load_gather(rows_v, [row, cnt + (step - 1)])
                    cnt = cnt + jnp.where(vals < q, step, 0)
                p = rid * _FINE + cnt
                base = ((p >> 7) << 10) + (p & (_BLK - 1))
                for comp in range(d):
                    idx_v[comp, pl.ds(j * _L, _L)] = base + comp * _BLK
                return c

            lax.fori_loop(0, vregs, fine_body, 0)

            copies = [
                pltpu.async_copy(parts_hbm.at[wid].at[idx_v.at[comp]],
                                 g_v.at[comp], sem)
                for comp in range(d)
            ]
            for c in copies:
                c.wait()

            def blk_body(b, c):
                for comp in range(d):
                    for cb in range(_BLK // _L):
                        out_v[b, comp, pl.ds(cb * _L, _L)] = (
                            g_v[comp, pl.ds(b * _BLK + cb * _L, _L)])
                return c

            lax.fori_loop(0, blk_chunk, blk_body, 0)

            pltpu.sync_copy(
                out_v, out_hbm.at[wid].at[pl.ds(ci * blk_chunk, blk_chunk)])
            return carry

        lax.fori_loop(0, n_chunks, chunk_body, 0)

    return resample_k


def kernel(particles, log_weights):
    t, n, d = particles.shape
    key = jax.random.key(42)
    keys = jax.random.split(key, t)

    def prep(lw, k):
        w = jnp.exp(lw)
        _, subkey = jax.random.split(k)
        p_cuml = jnp.cumsum(w)
        r = p_cuml[-1] * (1 - jax.random.uniform(subkey, (n,), dtype=p_cuml.dtype))
        return p_cuml, r

    p_cuml, r = jax.vmap(prep)(log_weights, keys)
    coarse = p_cuml[:, _FINE - 1::_FINE]
    cumrows = p_cuml.reshape(t, n // _FINE, _FINE)
    q = r.reshape(t, n // _L, _L)
    pview = particles.reshape(t, n // _BLK, _BLK, d).transpose(0, 1, 3, 2)
    pflat = pview.reshape(t, (n // _BLK) * d * _BLK)
    out4 = _make_resample(t, n, d)(coarse, q, cumrows, pflat)
    return out4.transpose(0, 1, 3, 2).reshape(t, n, d)


# speedup vs baseline: 1.8509x; 1.4556x over previous
import functools

import jax
import jax.numpy as jnp
from jax import lax
from jax.experimental import pallas as pl
from jax.experimental.pallas import tpu as pltpu
from jax.experimental.pallas import tpu_sc as plsc

_L = 16
_CHUNK = 2048
_FINE = 16
_COARSE_LVLS = 13
_FINE_LVLS = 4
_BLK = 128


def _make_resample(t_steps: int, n: int, d: int):
    n_coarse = n // _FINE
    n_chunks = n // _CHUNK
    vregs = _CHUNK // _L
    nblk = n // _BLK
    blk_chunk = _CHUNK // _BLK
    mesh = plsc.VectorSubcoreMesh(core_axis_name="c", subcore_axis_name="s")

    @functools.partial(
        pl.kernel,
        mesh=mesh,
        out_type=jax.ShapeDtypeStruct((t_steps, nblk, d, _BLK), jnp.float32),
        scratch_types=[
            pltpu.VMEM((n_coarse,), jnp.float32),
            pltpu.VMEM((_CHUNK // _L, _L), jnp.float32),
            pltpu.VMEM((_CHUNK,), jnp.int32),
            pltpu.VMEM((_CHUNK, _FINE), jnp.float32),
            pltpu.VMEM((d, _CHUNK), jnp.int32),
            pltpu.VMEM((d, _CHUNK), jnp.float32),
            pltpu.VMEM((blk_chunk, d, _BLK), jnp.float32),
            pltpu.SemaphoreType.DMA,
            pltpu.SemaphoreType.DMA,
        ],
        compiler_params=pltpu.CompilerParams(
            use_tc_tiling_on_sc=False, needs_layout_passes=False),
    )
    def resample_k(coarse_hbm, q_hbm, cumrows_hbm, parts_hbm, out_hbm,
                   coarse_v, q_v, rid_v, rows_v, idx_v, g_v, out_v,
                   sem_f, sem_c):
        wid = lax.axis_index("s") * 2 + lax.axis_index("c")
        iota = lax.broadcasted_iota(jnp.int32, (_L,), 0)

        pltpu.sync_copy(coarse_hbm.at[wid], coarse_v)

        def stage_search(ci):
            pltpu.sync_copy(
                q_hbm.at[wid].at[pl.ds(ci * (_CHUNK // _L), _CHUNK // _L)],
                q_v)

            def coarse_body(j, c):
                q = q_v[j]
                cnt = jnp.zeros((_L,), jnp.int32)
                for lvl in range(_COARSE_LVLS):
                    step = 1 << (_COARSE_LVLS - 1 - lvl)
                    vals = plsc.load_gather(coarse_v, [cnt + (step - 1)])
                    cnt = cnt + jnp.where(vals < q, step, 0)
                rid_v[pl.ds(j * _L, _L)] = cnt
                return c

            lax.fori_loop(0, vregs, coarse_body, 0)
            pltpu.async_copy(cumrows_hbm.at[wid].at[rid_v], rows_v, sem_f)

        def stage_fine(ci):
            pltpu.make_async_copy(
                cumrows_hbm.at[wid].at[rid_v], rows_v, sem_f).wait()

            def fine_body(j, c):
                q = q_v[j]
                rid = rid_v[pl.ds(j * _L, _L)]
                row = j * _L + iota
                cnt = jnp.zeros((_L,), jnp.int32)
                for lvl in range(_FINE_LVLS):
                    step = 1 << (_FINE_LVLS - 1 - lvl)
                    vals = plsc.load_gather(rows_v, [row, cnt + (step - 1)])
                    cnt = cnt + jnp.where(vals < q, step, 0)
                p = rid * _FINE + cnt
                base = ((p >> 7) << 10) + (p & (_BLK - 1))
                for comp in range(d):
                    idx_v[comp, pl.ds(j * _L, _L)] = base + comp * _BLK
                return c

            lax.fori_loop(0, vregs, fine_body, 0)
            for comp in range(d):
                pltpu.async_copy(parts_hbm.at[wid].at[idx_v.at[comp]],
                                 g_v.at[comp], sem_c)

        def stage_emit(ci):
            for comp in range(d):
                pltpu.make_async_copy(parts_hbm.at[wid].at[idx_v.at[comp]],
                                      g_v.at[comp], sem_c).wait()

            def blk_body(b, c):
                for comp in range(d):
                    for cb in range(_BLK // _L):
                        out_v[b, comp, pl.ds(cb * _L, _L)] = (
                            g_v[comp, pl.ds(b * _BLK + cb * _L, _L)])
                return c

            lax.fori_loop(0, blk_chunk, blk_body, 0)
            pltpu.sync_copy(
                out_v, out_hbm.at[wid].at[pl.ds(ci * blk_chunk, blk_chunk)])

        stage_search(0)
        stage_fine(0)

        def chunk_body(ci, carry):
            stage_search(ci)
            stage_emit(ci - 1)
            stage_fine(ci)
            return carry

        lax.fori_loop(1, n_chunks, chunk_body, 0)
        stage_emit(n_chunks - 1)

    return resample_k


def kernel(particles, log_weights):
    t, n, d = particles.shape
    key = jax.random.key(42)
    keys = jax.random.split(key, t)

    def prep(lw, k):
        w = jnp.exp(lw)
        _, subkey = jax.random.split(k)
        p_cuml = jnp.cumsum(w)
        r = p_cuml[-1] * (1 - jax.random.uniform(subkey, (n,), dtype=p_cuml.dtype))
        return p_cuml, r

    p_cuml, r = jax.vmap(prep)(log_weights, keys)
    coarse = p_cuml[:, _FINE - 1::_FINE]
    cumrows = p_cuml.reshape(t, n // _FINE, _FINE)
    q = r.reshape(t, n // _L, _L)
    pview = particles.reshape(t, n // _BLK, _BLK, d).transpose(0, 1, 3, 2)
    pflat = pview.reshape(t, (n // _BLK) * d * _BLK)
    out4 = _make_resample(t, n, d)(coarse, q, cumrows, pflat)
    return out4.transpose(0, 1, 3, 2).reshape(t, n, d)


# speedup vs baseline: 1.8709x; 1.0108x over previous
import functools

import jax
import jax.numpy as jnp
from jax import lax
from jax.experimental import pallas as pl
from jax.experimental.pallas import tpu as pltpu
from jax.experimental.pallas import tpu_sc as plsc

_L = 16
_CHUNK = 2048
_FINE = 16
_COARSE_LVLS = 13
_FINE_LVLS = 4
_BLK = 128


def _make_resample(t_steps: int, n: int, d: int):
    n_coarse = n // _FINE
    n_chunks = n // _CHUNK
    vregs = _CHUNK // _L
    nblk = n // _BLK
    blk_chunk = _CHUNK // _BLK
    mesh = plsc.VectorSubcoreMesh(core_axis_name="c", subcore_axis_name="s")

    @functools.partial(
        pl.kernel,
        mesh=mesh,
        out_type=jax.ShapeDtypeStruct((t_steps, nblk, d, _BLK), jnp.float32),
        scratch_types=[
            pltpu.VMEM((n_coarse,), jnp.float32),
            pltpu.VMEM((_CHUNK // _L, _L), jnp.float32),
            pltpu.VMEM((_CHUNK,), jnp.int32),
            pltpu.VMEM((_CHUNK, _FINE), jnp.float32),
            pltpu.VMEM((_CHUNK,), jnp.int32),
            pltpu.VMEM((d, _CHUNK), jnp.float32),
            pltpu.VMEM((blk_chunk, d, _BLK), jnp.float32),
            pltpu.SemaphoreType.DMA,
            pltpu.SemaphoreType.DMA,
        ],
        compiler_params=pltpu.CompilerParams(
            use_tc_tiling_on_sc=False, needs_layout_passes=False),
    )
    def resample_k(coarse_hbm, q_hbm, cumrows_hbm, parts_hbm, out_hbm,
                   coarse_v, q_v, rid_v, rows_v, idx_v, g_v, out_v,
                   sem_f, sem_c):
        wid = lax.axis_index("s") * 2 + lax.axis_index("c")
        iota = lax.broadcasted_iota(jnp.int32, (_L,), 0)

        pltpu.sync_copy(coarse_hbm.at[wid], coarse_v)

        def stage_search(ci):
            pltpu.sync_copy(
                q_hbm.at[wid].at[pl.ds(ci * (_CHUNK // _L), _CHUNK // _L)],
                q_v)

            def coarse_body(j4, c):
                qs = [q_v[j4 * 4 + u] for u in range(4)]
                cnts = [jnp.zeros((_L,), jnp.int32) for _ in range(4)]
                for lvl in range(_COARSE_LVLS):
                    step = 1 << (_COARSE_LVLS - 1 - lvl)
                    vals = [plsc.load_gather(coarse_v, [cnts[u] + (step - 1)])
                            for u in range(4)]
                    cnts = [cnts[u] + jnp.where(vals[u] < qs[u], step, 0)
                            for u in range(4)]
                for u in range(4):
                    rid_v[pl.ds((j4 * 4 + u) * _L, _L)] = cnts[u]
                return c

            lax.fori_loop(0, vregs // 4, coarse_body, 0)
            pltpu.async_copy(cumrows_hbm.at[wid].at[rid_v], rows_v, sem_f)

        def stage_fine(ci):
            pltpu.make_async_copy(
                cumrows_hbm.at[wid].at[rid_v], rows_v, sem_f).wait()

            def fine_body(j, c):
                q = q_v[j]
                rid = rid_v[pl.ds(j * _L, _L)]
                row = j * _L + iota
                cnt = jnp.zeros((_L,), jnp.int32)
                for lvl in range(_FINE_LVLS):
                    step = 1 << (_FINE_LVLS - 1 - lvl)
                    vals = plsc.load_gather(rows_v, [row, cnt + (step - 1)])
                    cnt = cnt + jnp.where(vals < q, step, 0)
                p = rid * _FINE + cnt
                idx_v[pl.ds(j * _L, _L)] = ((p >> 7) << 10) + (p & (_BLK - 1))
                return c

            lax.fori_loop(0, vregs, fine_body, 0)
            for comp in range(d):
                pltpu.async_copy(
                    parts_hbm.at[wid]
                    .at[pl.ds(comp * _BLK, (nblk - 1) * d * _BLK + _BLK)]
                    .at[idx_v],
                    g_v.at[comp], sem_c)

        def stage_emit(ci):
            for comp in range(d):
                pltpu.make_async_copy(
                    parts_hbm.at[wid]
                    .at[pl.ds(comp * _BLK, (nblk - 1) * d * _BLK + _BLK)]
                    .at[idx_v],
                    g_v.at[comp], sem_c).wait()

            def blk_body(b, c):
                for comp in range(d):
                    for cb in range(_BLK // _L):
                        out_v[b, comp, pl.ds(cb * _L, _L)] = (
                            g_v[comp, pl.ds(b * _BLK + cb * _L, _L)])
                return c

            lax.fori_loop(0, blk_chunk, blk_body, 0)
            pltpu.sync_copy(
                out_v, out_hbm.at[wid].at[pl.ds(ci * blk_chunk, blk_chunk)])

        stage_search(0)
        stage_fine(0)

        def chunk_body(ci, carry):
            stage_search(ci)
            stage_emit(ci - 1)
            stage_fine(ci)
            return carry

        lax.fori_loop(1, n_chunks, chunk_body, 0)
        stage_emit(n_chunks - 1)

    return resample_k


def kernel(particles, log_weights):
    t, n, d = particles.shape
    key = jax.random.key(42)
    keys = jax.random.split(key, t)

    def prep(lw, k):
        w = jnp.exp(lw)
        _, subkey = jax.random.split(k)
        p_cuml = jnp.cumsum(w)
        r = p_cuml[-1] * (1 - jax.random.uniform(subkey, (n,), dtype=p_cuml.dtype))
        return p_cuml, r

    p_cuml, r = jax.vmap(prep)(log_weights, keys)
    coarse = p_cuml[:, _FINE - 1::_FINE]
    cumrows = p_cuml.reshape(t, n // _FINE, _FINE)
    q = r.reshape(t, n // _L, _L)
    pview = particles.reshape(t, n // _BLK, _BLK, d).transpose(0, 1, 3, 2)
    pflat = pview.reshape(t, (n // _BLK) * d * _BLK)
    out4 = _make_resample(t, n, d)(coarse, q, cumrows, pflat)
    return out4.transpose(0, 1, 3, 2).reshape(t, n, d)


# speedup vs baseline: 2.1190x; 1.1326x over previous
import functools

import jax
import jax.numpy as jnp
from jax import lax
from jax.experimental import pallas as pl
from jax.experimental.pallas import tpu as pltpu
from jax.experimental.pallas import tpu_sc as plsc

_L = 16
_CHUNK = 2048
_FINE = 16
_COARSE_LVLS = 13
_FINE_LVLS = 4
_BLK = 128


def _make_resample(t_steps: int, n: int, d: int):
    n_coarse = n // _FINE
    n_chunks = n // _CHUNK
    vregs = _CHUNK // _L
    nblk = n // _BLK
    blk_chunk = _CHUNK // _BLK
    mesh = plsc.VectorSubcoreMesh(core_axis_name="c", subcore_axis_name="s")

    @functools.partial(
        pl.kernel,
        mesh=mesh,
        out_type=jax.ShapeDtypeStruct((t_steps, nblk, d, _BLK), jnp.float32),
        scratch_types=[
            pltpu.VMEM((n_coarse,), jnp.float32),
            pltpu.VMEM((_CHUNK // _L, _L), jnp.float32),
            pltpu.VMEM((_CHUNK,), jnp.int32),
            pltpu.VMEM((_CHUNK, _FINE), jnp.float32),
            pltpu.VMEM((2, _CHUNK), jnp.int32),
            pltpu.VMEM((2, d, _CHUNK), jnp.float32),
            pltpu.VMEM((blk_chunk, d, _BLK), jnp.float32),
            pltpu.SemaphoreType.DMA,
            pltpu.SemaphoreType.DMA,
            pltpu.SemaphoreType.DMA,
            pltpu.SemaphoreType.DMA,
        ],
        compiler_params=pltpu.CompilerParams(
            use_tc_tiling_on_sc=False, needs_layout_passes=False),
    )
    def resample_k(coarse_hbm, q_hbm, cumrows_hbm, parts_hbm, out_hbm,
                   coarse_v, q_v, rid_v, rows_v, idx_v, g_v, out_v,
                   sem_q, sem_f, sem_c0, sem_c1):
        wid = lax.axis_index("s") * 2 + lax.axis_index("c")
        iota = lax.broadcasted_iota(jnp.int32, (_L,), 0)

        pltpu.sync_copy(coarse_hbm.at[wid], coarse_v)

        def q_src(ci):
            return q_hbm.at[wid].at[pl.ds(ci * (_CHUNK // _L), _CHUNK // _L)]

        def stage_search(ci):
            pltpu.make_async_copy(q_src(ci), q_v, sem_q).wait()

            def coarse_body(j4, c):
                qs = [q_v[j4 * 4 + u] for u in range(4)]
                cnts = [jnp.zeros((_L,), jnp.int32) for _ in range(4)]
                for lvl in range(_COARSE_LVLS):
                    step = 1 << (_COARSE_LVLS - 1 - lvl)
                    vals = [plsc.load_gather(coarse_v, [cnts[u] + (step - 1)])
                            for u in range(4)]
                    cnts = [cnts[u] + jnp.where(vals[u] < qs[u], step, 0)
                            for u in range(4)]
                for u in range(4):
                    rid_v[pl.ds((j4 * 4 + u) * _L, _L)] = cnts[u]
                return c

            lax.fori_loop(0, vregs // 4, coarse_body, 0)
            pltpu.async_copy(cumrows_hbm.at[wid].at[rid_v], rows_v, sem_f)

        def comp_copy(par, comp, sem):
            return pltpu.make_async_copy(
                parts_hbm.at[wid]
                .at[pl.ds(comp * _BLK, (nblk - 1) * d * _BLK + _BLK)]
                .at[idx_v.at[par]],
                g_v.at[par].at[comp], sem)

        def stage_fine(ci, par, sem_c):
            pltpu.make_async_copy(
                cumrows_hbm.at[wid].at[rid_v], rows_v, sem_f).wait()

            def fine_body(j, c):
                q = q_v[j]
                rid = rid_v[pl.ds(j * _L, _L)]
                row = j * _L + iota
                cnt = jnp.zeros((_L,), jnp.int32)
                for lvl in range(_FINE_LVLS):
                    step = 1 << (_FINE_LVLS - 1 - lvl)
                    vals = plsc.load_gather(rows_v, [row, cnt + (step - 1)])
                    cnt = cnt + jnp.where(vals < q, step, 0)
                p = rid * _FINE + cnt
                idx_v[par, pl.ds(j * _L, _L)] = ((p >> 7) << 10) + (p & (_BLK - 1))
                return c

            lax.fori_loop(0, vregs, fine_body, 0)
            for comp in range(d):
                comp_copy(par, comp, sem_c).start()

        def stage_emit(ci, par, sem_c):
            for comp in range(d):
                comp_copy(par, comp, sem_c).wait()

            def blk_body(b, c):
                for comp in range(d):
                    for cb in range(_BLK // _L):
                        out_v[b, comp, pl.ds(cb * _L, _L)] = (
                            g_v[par, comp, pl.ds(b * _BLK + cb * _L, _L)])
                return c

            lax.fori_loop(0, blk_chunk, blk_body, 0)
            pltpu.sync_copy(
                out_v, out_hbm.at[wid].at[pl.ds(ci * blk_chunk, blk_chunk)])

        pltpu.async_copy(q_src(0), q_v, sem_q)
        stage_search(0)
        stage_fine(0, 0, sem_c0)
        pltpu.async_copy(q_src(1), q_v, sem_q)

        def body(ci2, carry):
            ci_a = ci2 * 2 + 1
            stage_search(ci_a)
            stage_fine(ci_a, 1, sem_c1)
            pltpu.async_copy(q_src(ci_a + 1), q_v, sem_q)
            stage_emit(ci_a - 1, 0, sem_c0)

            ci_b = ci_a + 1
            stage_search(ci_b)
            stage_fine(ci_b, 0, sem_c0)
            pltpu.async_copy(q_src(ci_b + 1), q_v, sem_q)
            stage_emit(ci_b - 1, 1, sem_c1)
            return carry

        lax.fori_loop(0, (n_chunks - 2) // 2, body, 0)

        ci_last = n_chunks - 1
        stage_search(ci_last)
        stage_fine(ci_last, 1, sem_c1)
        stage_emit(ci_last - 1, 0, sem_c0)
        stage_emit(ci_last, 1, sem_c1)

    return resample_k


def kernel(particles, log_weights):
    t, n, d = particles.shape
    key = jax.random.key(42)
    keys = jax.random.split(key, t)

    def prep(lw, k):
        w = jnp.exp(lw)
        _, subkey = jax.random.split(k)
        p_cuml = jnp.cumsum(w)
        r = p_cuml[-1] * (1 - jax.random.uniform(subkey, (n,), dtype=p_cuml.dtype))
        return p_cuml, r

    p_cuml, r = jax.vmap(prep)(log_weights, keys)
    coarse = p_cuml[:, _FINE - 1::_FINE]
    cumrows = p_cuml.reshape(t, n // _FINE, _FINE)
    q = r.reshape(t, n // _L, _L)
    pview = particles.reshape(t, n // _BLK, _BLK, d).transpose(0, 1, 3, 2)
    pflat = pview.reshape(t, (n // _BLK) * d * _BLK)
    out4 = _make_resample(t, n, d)(coarse, q, cumrows, pflat)
    return out4.transpose(0, 1, 3, 2).reshape(t, n, d)


# speedup vs baseline: 2.5271x; 1.1926x over previous
import functools

import jax
import jax.numpy as jnp
from jax import lax
from jax.experimental import pallas as pl
from jax.experimental.pallas import tpu as pltpu
from jax.experimental.pallas import tpu_sc as plsc

_L = 16
_CHUNK = 2048
_FINE = 16
_COARSE_LVLS = 13
_FINE_LVLS = 4
_BLK = 128
_ROT = (13, 15, 26, 6, 17, 29, 16, 24)


def _make_resample(t_steps: int, n: int, d: int):
    n_coarse = n // _FINE
    n_chunks = n // _CHUNK
    vregs = _CHUNK // _L
    nblk = n // _BLK
    blk_chunk = _CHUNK // _BLK
    mesh = plsc.VectorSubcoreMesh(core_axis_name="c", subcore_axis_name="s")

    @functools.partial(
        pl.kernel,
        mesh=mesh,
        out_type=jax.ShapeDtypeStruct((t_steps, nblk, d, _BLK), jnp.float32),
        scratch_types=[
            pltpu.VMEM((n_coarse,), jnp.float32),
            pltpu.VMEM((2, _L), jnp.uint32),
            pltpu.VMEM((_CHUNK // _L, _L), jnp.float32),
            pltpu.VMEM((_CHUNK,), jnp.int32),
            pltpu.VMEM((_CHUNK, _FINE), jnp.float32),
            pltpu.VMEM((2, _CHUNK), jnp.int32),
            pltpu.VMEM((2, d, _CHUNK), jnp.float32),
            pltpu.VMEM((blk_chunk, d, _BLK), jnp.float32),
            pltpu.SemaphoreType.DMA,
            pltpu.SemaphoreType.DMA,
            pltpu.SemaphoreType.DMA,
        ],
        compiler_params=pltpu.CompilerParams(
            use_tc_tiling_on_sc=False, needs_layout_passes=False),
    )
    def resample_k(coarse_hbm, keys_hbm, cumrows_hbm, parts_hbm, out_hbm,
                   coarse_v, kv, q_v, rid_v, rows_v, idx_v, g_v, out_v,
                   sem_f, sem_c0, sem_c1):
        wid = lax.axis_index("s") * 2 + lax.axis_index("c")
        iota = lax.broadcasted_iota(jnp.int32, (_L,), 0)

        pltpu.sync_copy(coarse_hbm.at[wid], coarse_v)
        pltpu.sync_copy(keys_hbm.at[wid], kv)
        total = plsc.load_gather(
            coarse_v, [jnp.full((_L,), n_coarse - 1, jnp.int32)])

        def stage_search(ci):
            k0 = kv[0]
            k1 = kv[1]
            ks2 = k0 ^ k1 ^ jnp.uint32(0x1BD11BDA)
            ks = (k0, k1, ks2)

            def qgen_body(j, c):
                cnt = (ci * _CHUNK + j * _L) + iota
                x0 = k0
                x1 = lax.convert_element_type(cnt, jnp.uint32) + k1
                for i in range(5):
                    rots = _ROT[4 * (i % 2):4 * (i % 2) + 4]
                    for r in rots:
                        x0 = x0 + x1
                        x1 = (jnp.left_shift(x1, jnp.uint32(r)) |
                              jnp.right_shift(x1, jnp.uint32(32 - r))) ^ x0
                    x0 = x0 + ks[(i + 1) % 3]
                    x1 = x1 + ks[(i + 2) % 3] + jnp.uint32(i + 1)
                bits = x0 ^ x1
                fbits = jnp.right_shift(bits, jnp.uint32(9)) | jnp.uint32(0x3F800000)
                u = lax.bitcast_convert_type(fbits, jnp.float32) - jnp.float32(1.0)
                q_v[j] = total * (jnp.float32(1.0) - u)
                return c

            lax.fori_loop(0, vregs, qgen_body, 0)

            def coarse_body(j4, c):
                qs = [q_v[j4 * 4 + u] for u in range(4)]
                cnts = [jnp.zeros((_L,), jnp.int32) for _ in range(4)]
                for lvl in range(_COARSE_LVLS):
                    step = 1 << (_COARSE_LVLS - 1 - lvl)
                    vals = [plsc.load_gather(coarse_v, [cnts[u] + (step - 1)])
                            for u in range(4)]
                    cnts = [cnts[u] + jnp.where(vals[u] < qs[u], step, 0)
                            for u in range(4)]
                for u in range(4):
                    rid_v[pl.ds((j4 * 4 + u) * _L, _L)] = cnts[u]
                return c

            lax.fori_loop(0, vregs // 4, coarse_body, 0)
            pltpu.async_copy(cumrows_hbm.at[wid].at[rid_v], rows_v, sem_f)

        def comp_copy(par, comp, sem):
            return pltpu.make_async_copy(
                parts_hbm.at[wid]
                .at[pl.ds(comp * _BLK, (nblk - 1) * d * _BLK + _BLK)]
                .at[idx_v.at[par]],
                g_v.at[par].at[comp], sem)

        def stage_fine(ci, par, sem_c):
            pltpu.make_async_copy(
                cumrows_hbm.at[wid].at[rid_v], rows_v, sem_f).wait()

            def fine_body(j, c):
                q = q_v[j]
                rid = rid_v[pl.ds(j * _L, _L)]
                row = j * _L + iota
                cnt = jnp.zeros((_L,), jnp.int32)
                for lvl in range(_FINE_LVLS):
                    step = 1 << (_FINE_LVLS - 1 - lvl)
                    vals = plsc.load_gather(rows_v, [row, cnt + (step - 1)])
                    cnt = cnt + jnp.where(vals < q, step, 0)
                p = rid * _FINE + cnt
                idx_v[par, pl.ds(j * _L, _L)] = ((p >> 7) << 10) + (p & (_BLK - 1))
                return c

            lax.fori_loop(0, vregs, fine_body, 0)
            for comp in range(d):
                comp_copy(par, comp, sem_c).start()

        def stage_emit(ci, par, sem_c):
            for comp in range(d):
                comp_copy(par, comp, sem_c).wait()

            def blk_body(b, c):
                for comp in range(d):
                    for cb in range(_BLK // _L):
                        out_v[b, comp, pl.ds(cb * _L, _L)] = (
                            g_v[par, comp, pl.ds(b * _BLK + cb * _L, _L)])
                return c

            lax.fori_loop(0, blk_chunk, blk_body, 0)
            pltpu.sync_copy(
                out_v, out_hbm.at[wid].at[pl.ds(ci * blk_chunk, blk_chunk)])

        stage_search(0)
        stage_fine(0, 0, sem_c0)

        def body(ci2, carry):
            ci_a = ci2 * 2 + 1
            stage_search(ci_a)
            stage_fine(ci_a, 1, sem_c1)
            stage_emit(ci_a - 1, 0, sem_c0)

            ci_b = ci_a + 1
            stage_search(ci_b)
            stage_fine(ci_b, 0, sem_c0)
            stage_emit(ci_b - 1, 1, sem_c1)
            return carry

        lax.fori_loop(0, (n_chunks - 2) // 2, body, 0)

        ci_last = n_chunks - 1
        stage_search(ci_last)
        stage_fine(ci_last, 1, sem_c1)
        stage_emit(ci_last - 1, 0, sem_c0)
        stage_emit(ci_last, 1, sem_c1)

    return resample_k


def kernel(particles, log_weights):
    t, n, d = particles.shape
    key = jax.random.key(42)
    keys = jax.random.split(key, t)

    p_cuml = jax.vmap(lambda lw: jnp.cumsum(jnp.exp(lw)))(log_weights)
    subkeys = jax.vmap(lambda k: jax.random.split(k)[1])(keys)
    keydata = jax.random.key_data(subkeys).astype(jnp.uint32)
    keys16 = jnp.broadcast_to(keydata[:, :, None], (t, 2, _L))

    coarse = p_cuml[:, _FINE - 1::_FINE]
    cumrows = p_cuml.reshape(t, n // _FINE, _FINE)
    pview = particles.reshape(t, n // _BLK, _BLK, d).transpose(0, 1, 3, 2)
    pflat = pview.reshape(t, (n // _BLK) * d * _BLK)
    out4 = _make_resample(t, n, d)(coarse, keys16, cumrows, pflat)
    return out4.transpose(0, 1, 3, 2).reshape(t, n, d)


# speedup vs baseline: 2.6897x; 1.0644x over previous
import functools

import jax
import jax.numpy as jnp
from jax import lax
from jax.experimental import pallas as pl
from jax.experimental.pallas import tpu as pltpu
from jax.experimental.pallas import tpu_sc as plsc

_L = 16
_CHUNK = 2048
_FINE = 16
_COARSE_LVLS = 13
_FINE_LVLS = 4
_BLK = 128
_ROT = (13, 15, 26, 6, 17, 29, 16, 24)


def _make_resample(t_steps: int, n: int, d: int):
    n_coarse = n // _FINE
    n_chunks = n // _CHUNK
    vregs = _CHUNK // _L
    nblk = n // _BLK
    blk_chunk = _CHUNK // _BLK
    mesh = plsc.VectorSubcoreMesh(core_axis_name="c", subcore_axis_name="s")

    @functools.partial(
        pl.kernel,
        mesh=mesh,
        out_type=jax.ShapeDtypeStruct((t_steps, nblk, d, _BLK), jnp.float32),
        scratch_types=[
            pltpu.VMEM((n_coarse,), jnp.float32),
            pltpu.VMEM((2, _L), jnp.uint32),
            pltpu.VMEM((_CHUNK // _L, _L), jnp.float32),
            pltpu.VMEM((_CHUNK,), jnp.int32),
            pltpu.VMEM((_CHUNK, _FINE), jnp.float32),
            pltpu.VMEM((2, _CHUNK), jnp.int32),
            pltpu.VMEM((2, d, _CHUNK), jnp.float32),
            pltpu.VMEM((blk_chunk, d, _BLK), jnp.float32),
            pltpu.SemaphoreType.DMA,
            pltpu.SemaphoreType.DMA,
            pltpu.SemaphoreType.DMA,
        ],
        compiler_params=pltpu.CompilerParams(
            use_tc_tiling_on_sc=False, needs_layout_passes=False),
    )
    def resample_k(coarse_hbm, keys_hbm, cumrows_hbm, parts_hbm, out_hbm,
                   coarse_v, kv, q_v, rid_v, rows_v, idx_v, g_v, out_v,
                   sem_f, sem_c0, sem_c1):
        wid = lax.axis_index("s") * 2 + lax.axis_index("c")
        iota = lax.broadcasted_iota(jnp.int32, (_L,), 0)

        pltpu.sync_copy(coarse_hbm.at[wid], coarse_v)
        pltpu.sync_copy(keys_hbm.at[wid], kv)
        total = plsc.load_gather(
            coarse_v, [jnp.full((_L,), n_coarse - 1, jnp.int32)])

        def stage_search(ci):
            k0 = kv[0]
            k1 = kv[1]
            ks2 = k0 ^ k1 ^ jnp.uint32(0x1BD11BDA)
            ks = (k0, k1, ks2)

            def qgen_body(j2, c):
                x0s = [k0, k0]
                x1s = []
                for u in range(2):
                    cnt = (ci * _CHUNK + (j2 * 2 + u) * _L) + iota
                    x1s.append(lax.convert_element_type(cnt, jnp.uint32) + k1)
                for i in range(5):
                    rots = _ROT[4 * (i % 2):4 * (i % 2) + 4]
                    for r in rots:
                        x0s = [x0s[u] + x1s[u] for u in range(2)]
                        x1s = [(jnp.left_shift(x1s[u], jnp.uint32(r)) |
                                jnp.right_shift(x1s[u], jnp.uint32(32 - r)))
                               ^ x0s[u] for u in range(2)]
                    x0s = [x0s[u] + ks[(i + 1) % 3] for u in range(2)]
                    x1s = [x1s[u] + ks[(i + 2) % 3] + jnp.uint32(i + 1)
                           for u in range(2)]
                for u in range(2):
                    bits = x0s[u] ^ x1s[u]
                    fbits = (jnp.right_shift(bits, jnp.uint32(9))
                             | jnp.uint32(0x3F800000))
                    uu = (lax.bitcast_convert_type(fbits, jnp.float32)
                          - jnp.float32(1.0))
                    q_v[j2 * 2 + u] = total * (jnp.float32(1.0) - uu)
                return c

            lax.fori_loop(0, vregs // 2, qgen_body, 0)

            def coarse_body(j4, c):
                qs = [q_v[j4 * 4 + u] for u in range(4)]
                cnts = [jnp.zeros((_L,), jnp.int32) for _ in range(4)]
                for lvl in range(_COARSE_LVLS):
                    step = 1 << (_COARSE_LVLS - 1 - lvl)
                    vals = [plsc.load_gather(coarse_v, [cnts[u] + (step - 1)])
                            for u in range(4)]
                    cnts = [cnts[u] + jnp.where(vals[u] < qs[u], step, 0)
                            for u in range(4)]
                for u in range(4):
                    rid_v[pl.ds((j4 * 4 + u) * _L, _L)] = cnts[u]
                return c

            lax.fori_loop(0, vregs // 4, coarse_body, 0)
            pltpu.async_copy(cumrows_hbm.at[wid].at[rid_v], rows_v, sem_f)

        def comp_copy(par, comp, sem):
            return pltpu.make_async_copy(
                parts_hbm.at[wid]
                .at[pl.ds(comp * _BLK, (nblk - 1) * d * _BLK + _BLK)]
                .at[idx_v.at[par]],
                g_v.at[par].at[comp], sem)

        def stage_fine(ci, par, sem_c):
            pltpu.make_async_copy(
                cumrows_hbm.at[wid].at[rid_v], rows_v, sem_f).wait()

            def fine_body(j2, c):
                qs = [q_v[j2 * 2 + u] for u in range(2)]
                rids = [rid_v[pl.ds((j2 * 2 + u) * _L, _L)] for u in range(2)]
                rows = [(j2 * 2 + u) * _L + iota for u in range(2)]
                cnts = [jnp.zeros((_L,), jnp.int32) for _ in range(2)]
                for lvl in range(_FINE_LVLS):
                    step = 1 << (_FINE_LVLS - 1 - lvl)
                    vals = [plsc.load_gather(rows_v,
                                             [rows[u], cnts[u] + (step - 1)])
                            for u in range(2)]
                    cnts = [cnts[u] + jnp.where(vals[u] < qs[u], step, 0)
                            for u in range(2)]
                for u in range(2):
                    p = rids[u] * _FINE + cnts[u]
                    idx_v[par, pl.ds((j2 * 2 + u) * _L, _L)] = (
                        ((p >> 7) << 10) + (p & (_BLK - 1)))
                return c

            lax.fori_loop(0, vregs // 2, fine_body, 0)
            for comp in range(d):
                comp_copy(par, comp, sem_c).start()

        def stage_emit(ci, par, sem_c):
            for comp in range(d):
                comp_copy(par, comp, sem_c).wait()

            def blk_body(b, c):
                for comp in range(d):
                    for cb in range(_BLK // _L):
                        out_v[b, comp, pl.ds(cb * _L, _L)] = (
                            g_v[par, comp, pl.ds(b * _BLK + cb * _L, _L)])
                return c

            lax.fori_loop(0, blk_chunk, blk_body, 0)
            pltpu.sync_copy(
                out_v, out_hbm.at[wid].at[pl.ds(ci * blk_chunk, blk_chunk)])

        stage_search(0)
        stage_fine(0, 0, sem_c0)

        def body(ci2, carry):
            ci_a = ci2 * 2 + 1
            stage_search(ci_a)
            stage_fine(ci_a, 1, sem_c1)
            stage_emit(ci_a - 1, 0, sem_c0)

            ci_b = ci_a + 1
            stage_search(ci_b)
            stage_fine(ci_b, 0, sem_c0)
            stage_emit(ci_b - 1, 1, sem_c1)
            return carry

        lax.fori_loop(0, (n_chunks - 2) // 2, body, 0)

        ci_last = n_chunks - 1
        stage_search(ci_last)
        stage_fine(ci_last, 1, sem_c1)
        stage_emit(ci_last - 1, 0, sem_c0)
        stage_emit(ci_last, 1, sem_c1)

    return resample_k


def kernel(particles, log_weights):
    t, n, d = particles.shape
    key = jax.random.key(42)
    keys = jax.random.split(key, t)

    p_cuml = jax.vmap(lambda lw: jnp.cumsum(jnp.exp(lw)))(log_weights)
    subkeys = jax.vmap(lambda k: jax.random.split(k)[1])(keys)
    keydata = jax.random.key_data(subkeys).astype(jnp.uint32)
    keys16 = jnp.broadcast_to(keydata[:, :, None], (t, 2, _L))

    coarse = p_cuml[:, _FINE - 1::_FINE]
    cumrows = p_cuml.reshape(t, n // _FINE, _FINE)
    pview = particles.reshape(t, n // _BLK, _BLK, d).transpose(0, 1, 3, 2)
    pflat = pview.reshape(t, (n // _BLK) * d * _BLK)
    out4 = _make_resample(t, n, d)(coarse, keys16, cumrows, pflat)
    return out4.transpose(0, 1, 3, 2).reshape(t, n, d)
